# Initial kernel scaffold; baseline (speedup 1.0000x reference)
import jax
import jax.numpy as jnp
from jax.experimental import pallas as pl


def kernel(solute_x, solute_edge_index, solute_batch, hydrated_solute_x, hydrated_solute_edge_index, hydrated_solute_batch, W_lx_s, b_lx_s, Wg_s, bg_s, W_lx_h, b_lx_h, Wg_h, bg_h, W_fc0, b_fc0, W_fc1, b_fc1, W_fc2, b_fc2):
    raise NotImplementedError("write your pallas kernel here")


# speedup vs baseline: 15.7396x; 15.7396x over previous
import functools
from typing import Any

import jax
import jax.numpy as jnp
from jax import lax
from jax.experimental import pallas as pl
from jax.experimental.pallas import tpu as pltpu
from jax.experimental.pallas import tpu_sc as plsc

N = 10000
E = 320000
D = 128
L = 3
G = 64

NC = 2
NS = 16
LANES = 16

CHUNK = 128
NCHUNKS = E // CHUNK
N_PAD = 10240
ROWS_PER_TILE = N_PAD // NS
BLK = 512
NBLK = N_PAD // BLK

@functools.lru_cache(maxsize=1)
def _get_mesh():
  return plsc.VectorSubcoreMesh(core_axis_name="c", subcore_axis_name="s",
                                num_cores=NC, num_subcores=NS)


def _tile_chunk_range(s):
  base = NCHUNKS // NS
  rem = NCHUNKS % NS
  start = s * base + jnp.minimum(s, rem)
  n = base + jnp.where(s < rem, 1, 0)
  return start, n


def _deg_body(dst2, zeros, ones, deg_out, acc, onesbuf, idxbuf):
  c = lax.axis_index("c")
  s = lax.axis_index("s")
  r0 = s * ROWS_PER_TILE
  pltpu.sync_copy(zeros.at[pl.ds(r0, ROWS_PER_TILE)],
                  acc.at[pl.ds(r0, ROWS_PER_TILE)])
  pltpu.sync_copy(ones, onesbuf)
  plsc.subcore_barrier()

  start, n = _tile_chunk_range(s)

  def body(i, carry):
    eb = (start + i) * CHUNK
    pltpu.sync_copy(dst2.at[c, pl.ds(eb, CHUNK)], idxbuf.at[0])
    pltpu.sync_copy(onesbuf, acc.at[idxbuf.at[0]], add=True)
    return carry

  lax.fori_loop(0, n, body, 0, unroll=False)
  plsc.subcore_barrier()
  pltpu.sync_copy(acc.at[pl.ds(r0, ROWS_PER_TILE)],
                  deg_out.at[c, pl.ds(r0, ROWS_PER_TILE)])


def _degree_counts(dst2, zeros, ones):
  return pl.kernel(
      _deg_body,
      out_type=jax.ShapeDtypeStruct((NC, N_PAD, D), jnp.float32),
      mesh=_get_mesh(),
      scratch_types=[
          pltpu.VMEM_SHARED((N_PAD, D), jnp.float32),
          pltpu.VMEM((CHUNK, D), jnp.float32),
          pltpu.VMEM((1, CHUNK), jnp.int32),
      ],
  )(dst2, zeros, ones)


def _edge_body(g, src2, dst2, zeros, s_out, acc, rows, sidx, didx, sem):
  c = lax.axis_index("c")
  s = lax.axis_index("s")
  r0 = s * ROWS_PER_TILE
  pltpu.sync_copy(zeros.at[pl.ds(r0, ROWS_PER_TILE)],
                  acc.at[pl.ds(r0, ROWS_PER_TILE)])
  plsc.subcore_barrier()

  start, n = _tile_chunk_range(s)

  def load_idx(i, buf):
    eb = (start + i) * CHUNK
    pltpu.sync_copy(src2.at[c, pl.ds(eb, CHUNK)], sidx.at[buf])
    pltpu.sync_copy(dst2.at[c, pl.ds(eb, CHUNK)], didx.at[buf])

  def fire_gather(buf):
    return pltpu.async_copy(g.at[sidx.at[buf]], rows.at[buf], sem.at[buf])

  @pl.when(n > 0)
  def _prologue():
    load_idx(0, 0)
    fire_gather(0)

  def body(i, carry):
    buf = lax.rem(i, 2)
    nbuf = 1 - buf

    @pl.when(i + 1 < n)
    def _fire_next():
      load_idx(i + 1, nbuf)
      fire_gather(nbuf)

    pltpu.make_async_copy(g.at[sidx.at[buf]], rows.at[buf], sem.at[buf]).wait()
    pltpu.sync_copy(rows.at[buf], acc.at[didx.at[buf]], add=True)
    return carry

  lax.fori_loop(0, n, body, 0, unroll=False)
  plsc.subcore_barrier()
  pltpu.sync_copy(acc.at[pl.ds(r0, ROWS_PER_TILE)],
                  s_out.at[c, pl.ds(r0, ROWS_PER_TILE)])


def _edge_segment_sum(g_flat, src2, dst2, zeros):
  return pl.kernel(
      _edge_body,
      out_type=jax.ShapeDtypeStruct((NC, N_PAD, D), jnp.float32),
      mesh=_get_mesh(),
      scratch_types=[
          pltpu.VMEM_SHARED((N_PAD, D), jnp.float32),
          pltpu.VMEM((2, CHUNK, D), jnp.float32),
          pltpu.VMEM((2, CHUNK), jnp.int32),
          pltpu.VMEM((2, CHUNK), jnp.int32),
          pltpu.SemaphoreType.DMA((2,)),
      ],
  )(g_flat, src2, dst2, zeros)


def _k0_body(x, deg, wlx, blx, wg, bg, h0_out, g0_out):
  dinv = lax.rsqrt(deg[0, :, 0:1] + 1.0)
  h0 = jnp.dot(x[0], wlx[0], preferred_element_type=jnp.float32) + blx[0]
  hw = jnp.dot(h0, wg[0], preferred_element_type=jnp.float32) + bg[0]
  h0_out[0] = h0
  g0_out[0] = dinv * hw


def _k_mid_body(sagg, gp, hp, deg, wg, bg, h_out, g_out):
  dinv = lax.rsqrt(deg[0, :, 0:1] + 1.0)
  h = jnp.maximum(dinv * (sagg[0] + gp[0]) + hp[0], 0.0)
  hw = jnp.dot(h, wg[0], preferred_element_type=jnp.float32) + bg[0]
  h_out[0] = h
  g_out[0] = dinv * hw


def _k3_body(sagg, gp, hp, deg, batch, wf0, bf0, wf1, bf1, wf2, bf2,
             out, acc):
  b = pl.program_id(0)
  r = pl.program_id(1)

  @pl.when(jnp.logical_and(b == 0, r == 0))
  def _():
    acc[...] = jnp.zeros_like(acc)

  dinv = lax.rsqrt(deg[0, :, 0:1] + 1.0)
  h3 = dinv * (sagg[0] + gp[0]) + hp[0]
  ids = batch[0, 0, 0]
  gids = lax.broadcasted_iota(jnp.int32, (G, BLK), 0)
  onehot = (ids[None, :] == gids).astype(jnp.float32)
  acc[...] += jnp.dot(onehot, h3, preferred_element_type=jnp.float32)

  @pl.when(jnp.logical_and(b == NC - 1, r == NBLK - 1))
  def _():
    rep = acc[...]
    t = jnp.maximum(jnp.dot(rep, wf0[...],
                            preferred_element_type=jnp.float32) + bf0[...], 0.0)
    t = jnp.maximum(jnp.dot(t, wf1[...],
                            preferred_element_type=jnp.float32) + bf1[...], 0.0)
    out[...] = jnp.dot(t, wf2[...],
                       preferred_element_type=jnp.float32) + bf2[...]


def _row_spec(last):
  return pl.BlockSpec((1, BLK, last), lambda b, r: (b, r, 0))


def _wspec():
  return pl.BlockSpec((1, D, D), lambda b, r: (b, 0, 0))


def _bspec():
  return pl.BlockSpec((1, 1, D), lambda b, r: (b, 0, 0))


def _tc_layer0(x2, deg16, wlx2, blx2, wg2_0, bg2_0):
  return pl.pallas_call(
      _k0_body,
      grid=(NC, NBLK),
      in_specs=[_row_spec(D), _row_spec(D), _wspec(), _bspec(),
                _wspec(), _bspec()],
      out_specs=[_row_spec(D), _row_spec(D)],
      out_shape=[jax.ShapeDtypeStruct((NC, N_PAD, D), jnp.float32)] * 2,
  )(x2, deg16, wlx2, blx2, wg2_0, bg2_0)


def _tc_layer_mid(sagg, gp, hp, deg16, wg2_l, bg2_l):
  return pl.pallas_call(
      _k_mid_body,
      grid=(NC, NBLK),
      in_specs=[_row_spec(D), _row_spec(D), _row_spec(D), _row_spec(D),
                _wspec(), _bspec()],
      out_specs=[_row_spec(D), _row_spec(D)],
      out_shape=[jax.ShapeDtypeStruct((NC, N_PAD, D), jnp.float32)] * 2,
  )(sagg, gp, hp, deg16, wg2_l, bg2_l)


def _tc_final(sagg, gp, hp, deg16, batch4, wf0, bf0, wf1, bf1, wf2, bf2):
  full = lambda shp: pl.BlockSpec(shp, lambda b, r: (0,) * len(shp))
  return pl.pallas_call(
      _k3_body,
      grid=(NC, NBLK),
      in_specs=[_row_spec(D), _row_spec(D), _row_spec(D), _row_spec(D),
                pl.BlockSpec((1, 1, 1, BLK), lambda b, r: (b, r, 0, 0)),
                full((D, 128)), full((1, 128)),
                full((128, G)), full((1, G)),
                full((G, 1)), full((1, 1))],
      out_specs=full((G, 1)),
      out_shape=jax.ShapeDtypeStruct((G, 1), jnp.float32),
      scratch_shapes=[pltpu.VMEM((G, D), jnp.float32)],
  )(sagg, gp, hp, deg16, batch4, wf0, bf0, wf1, bf1, wf2, bf2)


def kernel(solute_x, solute_edge_index, solute_batch, hydrated_solute_x,
           hydrated_solute_edge_index, hydrated_solute_batch, W_lx_s, b_lx_s,
           Wg_s, bg_s, W_lx_h, b_lx_h, Wg_h, bg_h, W_fc0, b_fc0, W_fc1, b_fc1,
           W_fc2, b_fc2):
  f32 = jnp.float32
  pad_rows = N_PAD - N

  x2 = jnp.pad(jnp.stack([solute_x, hydrated_solute_x]),
               ((0, 0), (0, pad_rows), (0, 0)))
  src2 = jnp.stack([solute_edge_index[0], hydrated_solute_edge_index[0]])
  src2 = src2 + (jnp.arange(NC, dtype=jnp.int32) * N_PAD)[:, None]
  dst2 = jnp.stack([solute_edge_index[1], hydrated_solute_edge_index[1]])
  batch2 = jnp.pad(jnp.stack([solute_batch, hydrated_solute_batch]),
                   ((0, 0), (0, pad_rows)), constant_values=G)
  batch4 = batch2.reshape(NC, NBLK, 1, BLK)

  zeros = jnp.zeros((N_PAD, D), f32)
  ones = jnp.ones((CHUNK, D), f32)

  wlx2 = jnp.stack([W_lx_s, W_lx_h])
  blx2 = jnp.stack([b_lx_s, b_lx_h])[:, None, :]
  wg2 = jnp.stack([Wg_s, Wg_h])
  bg2 = jnp.stack([bg_s, bg_h])[:, :, None, :]

  deg16 = _degree_counts(dst2, zeros, ones)

  h, g = _tc_layer0(x2, deg16, wlx2, blx2, wg2[:, 0], bg2[:, 0])
  for l in range(1, L):
    s_agg = _edge_segment_sum(g.reshape(NC * N_PAD, D), src2, dst2, zeros)
    h, g = _tc_layer_mid(s_agg, g, h, deg16, wg2[:, l], bg2[:, l])
  s_agg = _edge_segment_sum(g.reshape(NC * N_PAD, D), src2, dst2, zeros)

  out = _tc_final(s_agg, g, h, deg16, batch4,
                  W_fc0, b_fc0[None, :], W_fc1, b_fc1[None, :],
                  W_fc2, b_fc2[None, :])
  return out
